# Initial kernel scaffold; baseline (speedup 1.0000x reference)
import jax
import jax.numpy as jnp
from jax.experimental import pallas as pl


def kernel(x, edge_index, W1, a_src1, a_dst1, b1, W2, a_src2, a_dst2, b2):
    raise NotImplementedError("write your pallas kernel here")


# speedup vs baseline: 11.3033x; 11.3033x over previous
import jax
import jax.numpy as jnp
from jax import lax
from jax.experimental import pallas as pl
from jax.experimental.pallas import tpu as pltpu
from jax.experimental.pallas import tpu_sc as plsc

N_NODES = 10000
D = 128
HD = D // 2
L = 16
NT = 10240
NBLK = NT // D
E_BASE = 320000
E_TOT = E_BASE + N_NODES
CH = 64
CPT = 2 * -(-E_TOT // (16 * CH * 2))
E_PAD = CPT * 16 * CH
CPW = CPT // 2
RPT = 632
RPT_TAIL = N_NODES - 15 * RPT
NFLAT = 16 * RPT
RB = -(-RPT // 64)


def _mm_at(x, w):
  return lax.dot_general(x, w, (((1,), (1,)), ((), ())),
                         preferred_element_type=jnp.float32)


def _emit_tables(h, asv, adv, t0_ref, t1_ref, tdst_ref, als_ref, ald_ref):
  als = jnp.sum(h * asv, axis=1, keepdims=True)
  row = pl.program_id(0) * D + lax.broadcasted_iota(jnp.int32, (D, 1), 0)
  als = jnp.where(row >= N_NODES, -1e30, als)
  ald = jnp.sum(h * adv, axis=1, keepdims=True)
  als16 = jnp.broadcast_to(als, (D, L))
  zpad = jnp.zeros((D, D - HD - L), jnp.float32)
  t0_ref[...] = jnp.concatenate([h[:, :HD], als16, zpad], axis=1)
  t1_ref[...] = jnp.concatenate([h[:, HD:], als16, zpad], axis=1)
  tdst_ref[...] = jnp.concatenate(
      [jnp.broadcast_to(ald, (D, L)), jnp.zeros((D, D - L), jnp.float32)],
      axis=1)
  als_ref[...] = als
  ald_ref[...] = ald


def _stage_in_body(x_ref, w_ref, asv_ref, adv_ref,
                   t0_ref, t1_ref, tdst_ref, als_ref, ald_ref):
  h = _mm_at(x_ref[...], w_ref[...])
  _emit_tables(h, asv_ref[...], adv_ref[...],
               t0_ref, t1_ref, tdst_ref, als_ref, ald_ref)


def _stage_mid_body(acc_ref, d0_ref, d1_ref, b_ref, w_ref,
                    asv_ref, adv_ref,
                    t0_ref, t1_ref, tdst_ref, als_ref, ald_ref):
  den = d0_ref[...] + d1_ref[...] + 1e-16
  z = jnp.maximum(acc_ref[...] / den + b_ref[...], 0.0)
  h = _mm_at(z, w_ref[...])
  _emit_tables(h, asv_ref[...], adv_ref[...],
               t0_ref, t1_ref, tdst_ref, als_ref, ald_ref)


def _stage_out_body(acc_ref, d0_ref, d1_ref, b_ref, o_ref):
  den = d0_ref[...] + d1_ref[...] + 1e-16
  o = acc_ref[...] / den + b_ref[...]
  m = jnp.max(o, axis=1, keepdims=True)
  ex = jnp.exp(o - m)
  o_ref[...] = o - m - jnp.log(jnp.sum(ex, axis=1, keepdims=True))


_BLK = lambda r, c: pl.BlockSpec((r, c), lambda i: (i, 0))
_REP = lambda r, c: pl.BlockSpec((r, c), lambda i: (0, 0))

_TAB_OUT_SPECS = [_BLK(D, D), _BLK(D, D), _BLK(D, D), _BLK(D, 1), _BLK(D, 1)]
_TAB_OUT_SHAPE = [jax.ShapeDtypeStruct((NT, D), jnp.float32),
                  jax.ShapeDtypeStruct((NT, D), jnp.float32),
                  jax.ShapeDtypeStruct((NT, D), jnp.float32),
                  jax.ShapeDtypeStruct((NT, 1), jnp.float32),
                  jax.ShapeDtypeStruct((NT, 1), jnp.float32)]


def _stage_in(x, w, a_s, a_d):
  return pl.pallas_call(
      _stage_in_body,
      grid=(NBLK,),
      in_specs=[_BLK(D, D), _REP(D, D), _REP(1, D), _REP(1, D)],
      out_specs=_TAB_OUT_SPECS,
      out_shape=_TAB_OUT_SHAPE,
  )(x, w, a_s.reshape(1, D), a_d.reshape(1, D))


def _stage_mid(acc, den0, den1, b, w, a_s, a_d):
  return pl.pallas_call(
      _stage_mid_body,
      grid=(NBLK,),
      in_specs=[_BLK(D, D), _BLK(D, 1), _BLK(D, 1),
                _REP(1, D), _REP(D, D), _REP(1, D), _REP(1, D)],
      out_specs=_TAB_OUT_SPECS,
      out_shape=_TAB_OUT_SHAPE,
  )(acc, den0, den1, b.reshape(1, D), w,
    a_s.reshape(1, D), a_d.reshape(1, D))


def _stage_out(acc, den0, den1, b):
  return pl.pallas_call(
      _stage_out_body,
      grid=(NBLK,),
      in_specs=[_BLK(D, D), _BLK(D, 1), _BLK(D, 1), _REP(1, D)],
      out_specs=_BLK(D, D),
      out_shape=jax.ShapeDtypeStruct((NT, D), jnp.float32),
  )(acc, den0, den1, b.reshape(1, D))


def _over_tile_rows(piece, s):
  for k in range(RPT // 128):
    piece(k * 128, 128)
  tail = (RPT // 128) * 128
  @pl.when(s < 15)
  def _():
    piece(tail, RPT - tail)
  @pl.when(s == 15)
  def _():
    piece(tail, RPT_TAIL - tail)


def _acc_body(gidx_hbm, dst_hbm, ridx_hbm, t3_hbm, accp_hbm,
              dst_v, idx1_v, hrow_v, rows_v, zrow_v, wrow_v, ridx_v,
              acc_sp, sem_r):
  c = lax.axis_index("c")
  s = lax.axis_index("s")

  pltpu.sync_copy(dst_hbm.at[s], dst_v)
  pltpu.sync_copy(ridx_hbm.at[s], ridx_v)

  def _zrow(i, carry):
    for j in range(HD // L):
      zrow_v[i, pl.ds(j * L, L)] = jnp.zeros((L,), jnp.float32)
    return carry
  lax.fori_loop(0, 64, _zrow, 0)
  for b in range(RB):
    pltpu.sync_copy(zrow_v, acc_sp.at[ridx_v.at[pl.ds(b * 64, 64)]])
  plsc.subcore_barrier()

  def _chunk(ci, carry):
    pltpu.sync_copy(gidx_hbm.at[c, s, ci], idx1_v)
    cp_r = pltpu.async_copy(t3_hbm.at[idx1_v], hrow_v, sem_r)
    cp_r.wait()
    def _srow(i, carry2):
      e = hrow_v[i, pl.ds(HD, L)] + hrow_v[CH + i, pl.ds(0, L)]
      e = jnp.maximum(e, e * 0.2)
      w = jnp.exp(e)
      for j in range(HD // L):
        rows_v[i, pl.ds(j * L, L)] = hrow_v[i, pl.ds(j * L, L)] * w
      return carry2
    lax.fori_loop(0, CH, _srow, 0)
    pltpu.sync_copy(rows_v, acc_sp.at[dst_v.at[pl.ds(ci * CH, CH)]], add=True)
    return carry
  lax.fori_loop(0, CPT, _chunk, 0)

  plsc.subcore_barrier()
  for b in range(RB):
    pltpu.async_copy(acc_sp.at[ridx_v.at[pl.ds(b * 64, 64)]],
                     zrow_v, sem_r).wait()
    def _xrow(i, carry):
      for j in range(HD // L):
        wrow_v[i, pl.ds(j * L, L)] = zrow_v[i, pl.ds(j * L, L)]
      return carry
    lax.fori_loop(0, 64, _xrow, 0)
    pltpu.sync_copy(wrow_v, accp_hbm.at[c].at[ridx_v.at[pl.ds(b * 64, 64)]])


def _edge_acc(gidx, dst3d, ridx, t0, t1, tdst):
  t3 = jnp.concatenate([t0, t1, tdst], axis=0)
  mesh = plsc.VectorSubcoreMesh(core_axis_name="c", subcore_axis_name="s")
  return pl.kernel(
      _acc_body,
      out_type=jax.ShapeDtypeStruct((2, NFLAT, D), jnp.float32),
      mesh=mesh,
      scratch_types=[
          pltpu.VMEM((CPT * CH,), jnp.int32),
          pltpu.VMEM((2 * CH,), jnp.int32),
          pltpu.VMEM((2 * CH, D), jnp.float32),
          pltpu.VMEM((CH, HD), jnp.float32),
          pltpu.VMEM((64, HD), jnp.float32),
          pltpu.VMEM((64, D), jnp.float32),
          pltpu.VMEM((RB * 64,), jnp.int32),
          pltpu.VMEM_SHARED((N_NODES, HD), jnp.float32),
          pltpu.SemaphoreType.DMA,
      ],
  )(gidx, dst3d, ridx, t3)


def _den_body(src_hbm, dst_hbm, als_hbm, ald_hbm, denp_hbm,
              src_v, dst_v, alsc_v, aldc_v, w_v, tb_v,
              als_sp, ald_sp, den_sp, sem_a, sem_b):
  c = lax.axis_index("c")
  s = lax.axis_index("s")

  pltpu.sync_copy(src_hbm.at[s], src_v)
  pltpu.sync_copy(dst_hbm.at[s], dst_v)
  trows = pl.ds(s * (NT // 16), NT // 16)
  pltpu.sync_copy(als_hbm.at[trows], tb_v)
  pltpu.sync_copy(tb_v, als_sp.at[trows])
  pltpu.sync_copy(ald_hbm.at[trows], tb_v)
  pltpu.sync_copy(tb_v, ald_sp.at[trows])

  for j in range(128 // L):
    tb_v[pl.ds(j * L, L)] = jnp.zeros((L,), jnp.float32)
  base = s * RPT
  _over_tile_rows(lambda off, nr: pltpu.sync_copy(
      tb_v.at[pl.ds(0, nr)], den_sp.at[pl.ds(base + off, nr)]), s)
  plsc.subcore_barrier()

  def _chunk(ci, carry):
    row = (c * CPW + ci) * CH
    cp_a = pltpu.async_copy(als_sp.at[src_v.at[pl.ds(row, CH)]], alsc_v, sem_a)
    cp_b = pltpu.async_copy(ald_sp.at[dst_v.at[pl.ds(row, CH)]], aldc_v, sem_b)
    cp_a.wait()
    cp_b.wait()
    def _grp(g, carry2):
      e = alsc_v[pl.ds(g * L, L)] + aldc_v[pl.ds(g * L, L)]
      e = jnp.maximum(e, e * 0.2)
      w_v[pl.ds(g * L, L)] = jnp.exp(e)
      return carry2
    lax.fori_loop(0, CH // L, _grp, 0)
    pltpu.sync_copy(w_v, den_sp.at[dst_v.at[pl.ds(row, CH)]], add=True)
    return carry
  lax.fori_loop(0, CPW, _chunk, 0)

  plsc.subcore_barrier()
  def _wout(off, nr):
    pltpu.sync_copy(den_sp.at[pl.ds(base + off, nr)], tb_v.at[pl.ds(0, nr)])
    pltpu.sync_copy(tb_v.at[pl.ds(0, nr)], denp_hbm.at[c, s, pl.ds(off, nr)])
  _over_tile_rows(_wout, s)


def _edge_den(src3d, dst3d, als, ald):
  mesh = plsc.VectorSubcoreMesh(core_axis_name="c", subcore_axis_name="s")
  return pl.kernel(
      _den_body,
      out_type=jax.ShapeDtypeStruct((2, 16, 640), jnp.float32),
      mesh=mesh,
      scratch_types=[
          pltpu.VMEM((CPT * CH,), jnp.int32),
          pltpu.VMEM((CPT * CH,), jnp.int32),
          pltpu.VMEM((CH,), jnp.float32),
          pltpu.VMEM((CH,), jnp.float32),
          pltpu.VMEM((CH,), jnp.float32),
          pltpu.VMEM((NT // 16,), jnp.float32),
          pltpu.VMEM_SHARED((NT,), jnp.float32),
          pltpu.VMEM_SHARED((NT,), jnp.float32),
          pltpu.VMEM_SHARED((N_NODES,), jnp.float32),
          pltpu.SemaphoreType.DMA,
          pltpu.SemaphoreType.DMA,
      ],
  )(src3d, dst3d, als, ald)


def kernel(x, edge_index, W1, a_src1, a_dst1, b1, W2, a_src2, a_dst2, b2):
  n = x.shape[0]
  loop = jnp.arange(n, dtype=edge_index.dtype)
  pad = E_PAD - E_TOT
  src = jnp.concatenate([edge_index[0], loop,
                         jnp.full((pad,), n, edge_index.dtype)])
  dst = jnp.concatenate([edge_index[1], loop,
                         jnp.zeros((pad,), edge_index.dtype)])
  src3d = src.reshape(16, CPT, CH)
  dst3d = dst.reshape(16, CPT, CH)
  dst2 = jnp.stack([dst3d, dst3d]) + 2 * NT
  gidx = jnp.concatenate([jnp.stack([src3d, src3d + NT]), dst2], axis=3)
  srcf = src.reshape(16, CPT * CH).astype(jnp.int32)
  dstf = dst.reshape(16, CPT * CH).astype(jnp.int32)
  xp = jnp.pad(x, ((0, NT - n), (0, 0)))

  rowid = jnp.arange(16)[:, None, None] * RPT + \
      jnp.arange(RB)[None, :, None] * 64 + jnp.arange(64)[None, None, :]
  ridx = jnp.minimum(rowid, N_NODES - 1).astype(jnp.int32).reshape(16, RB * 64)

  def _assemble(accp, denp):
    af = accp[:, :N_NODES, :HD]
    acc = jnp.pad(jnp.concatenate([af[0], af[1]], axis=1),
                  ((0, NT - N_NODES), (0, 0)))
    df = denp[:, :, :RPT].reshape(2, NFLAT)[:, :N_NODES]
    d0 = jnp.pad(df[0], (0, NT - N_NODES)).reshape(NT, 1)
    d1 = jnp.pad(df[1], (0, NT - N_NODES)).reshape(NT, 1)
    return acc, d0, d1

  t01, t11, tdst1, als1, ald1 = _stage_in(xp, W1, a_src1, a_dst1)
  accp1 = _edge_acc(gidx, dstf, ridx, t01, t11, tdst1)
  denp1 = _edge_den(srcf, dstf, als1.reshape(NT), ald1.reshape(NT))
  acc, d0, d1 = _assemble(accp1, denp1)
  t02, t12, tdst2, als2, ald2 = _stage_mid(acc, d0, d1, b1, W2,
                                           a_src2, a_dst2)
  accp2 = _edge_acc(gidx, dstf, ridx, t02, t12, tdst2)
  denp2 = _edge_den(srcf, dstf, als2.reshape(NT), ald2.reshape(NT))
  acc, d0, d1 = _assemble(accp2, denp2)
  out = _stage_out(acc, d0, d1, b2)
  return out[:n]
